# baseline (speedup 1.0000x reference)
import functools

import jax
import jax.numpy as jnp
from jax import lax
from jax.experimental import pallas as pl
from jax.experimental.pallas import tpu as pltpu
from jax.experimental.pallas import tpu_sc as plsc

N = 10000
E = 320000
D = 128
R = 8
B = 4
H = 128
RP = R + 1

NC = 2
NS = 16
NW = NC * NS
EPW = E // NW
C = 80
NCH = EPW // C
G = 5
NGRP = NCH // G
NBUF = 4
DB = 80
NDB = N // DB
KDR = -(-NDB // NS)

BN = 1000


def _mk_w(coef_ref, basis_ref, r):
  w = coef_ref[r, 0] * basis_ref[0]
  for b in range(1, B):
    w = w + coef_ref[r, b] * basis_ref[b]
  return w


def _project1_body(coef_ref, basis_ref, loop_ref, x_ref, out_ref):
  xb = x_ref[...]
  for r in range(R):
    w = _mk_w(coef_ref, basis_ref, r)
    out_ref[r] = jnp.dot(xb, w, preferred_element_type=jnp.float32)
  out_ref[R] = jnp.dot(xb, loop_ref[...], preferred_element_type=jnp.float32)


def _project1(x, coef1, basis1, loop1):
  return pl.pallas_call(
      _project1_body,
      grid=(N // BN,),
      in_specs=[
          pl.BlockSpec(memory_space=pltpu.SMEM),
          pl.BlockSpec((B, D, H), lambda i: (0, 0, 0)),
          pl.BlockSpec((D, H), lambda i: (0, 0)),
          pl.BlockSpec((BN, D), lambda i: (i, 0)),
      ],
      out_specs=pl.BlockSpec((RP, BN, H), lambda i: (0, i, 0)),
      out_shape=jax.ShapeDtypeStruct((RP, N, H), jnp.float32),
  )(coef1, basis1, loop1, x)


def _finish1_project2_body(coef_ref, basis_ref, loop_ref, p_ref, t1s_ref,
                           bias_ref, out_ref):
  h = p_ref[0] + p_ref[1] + t1s_ref[0] + bias_ref[...]
  h = jnp.where(h >= 0, h, h * jnp.float32(0.01))
  for r in range(R):
    w = _mk_w(coef_ref, basis_ref, r)
    out_ref[r] = jnp.dot(h, w, preferred_element_type=jnp.float32)
  out_ref[R] = jnp.dot(h, loop_ref[...], preferred_element_type=jnp.float32)


def _finish1_project2(p1, t1, bias1, coef2, basis2, loop2):
  return pl.pallas_call(
      _finish1_project2_body,
      grid=(N // BN,),
      in_specs=[
          pl.BlockSpec(memory_space=pltpu.SMEM),
          pl.BlockSpec((B, H, H), lambda i: (0, 0, 0)),
          pl.BlockSpec((H, H), lambda i: (0, 0)),
          pl.BlockSpec((NC, BN, H), lambda i: (0, i, 0)),
          pl.BlockSpec((1, BN, H), lambda i: (R, i, 0)),
          pl.BlockSpec((1, H), lambda i: (0, 0)),
      ],
      out_specs=pl.BlockSpec((RP, BN, H), lambda i: (0, i, 0)),
      out_shape=jax.ShapeDtypeStruct((RP, N, H), jnp.float32),
  )(coef2, basis2, loop2, p1, t1, bias1)


def _finish2_body(p_ref, t2s_ref, bias_ref, out_ref):
  out_ref[...] = p_ref[0] + p_ref[1] + t2s_ref[0] + bias_ref[...]


def _finish2(p2, t2, bias2):
  return pl.pallas_call(
      _finish2_body,
      grid=(N // BN,),
      in_specs=[
          pl.BlockSpec((NC, BN, H), lambda i: (0, i, 0)),
          pl.BlockSpec((1, BN, H), lambda i: (R, i, 0)),
          pl.BlockSpec((1, H), lambda i: (0, 0)),
      ],
      out_specs=pl.BlockSpec((BN, H), lambda i: (i, 0)),
      out_shape=jax.ShapeDtypeStruct((N, H), jnp.float32),
  )(p2, t2, bias2)


def _sc_scatter_body(table_hbm, gidx_hbm, didx_hbm, out_hbm,
                     gidx_v, didx_v, rows_v, acc_sh, isem, gsem):
  cid = lax.axis_index("c")
  sid = lax.axis_index("s")
  wid = sid * NC + cid

  pltpu.async_copy(gidx_hbm.at[wid, 0], gidx_v.at[0], isem)
  pltpu.async_copy(didx_hbm.at[wid, 0], didx_v.at[0], isem)
  pltpu.make_async_copy(gidx_hbm.at[wid, 0], gidx_v.at[0], isem).wait()
  pltpu.make_async_copy(didx_hbm.at[wid, 0], didx_v.at[0], isem).wait()
  for b in range(NBUF - 1):
    pltpu.async_copy(table_hbm.at[gidx_v.at[0, b]], rows_v.at[b],
                     gsem.at[b])

  zeros16 = jnp.zeros((16,), jnp.float32)

  def _zero_row(i, carry):
    for k in range(H // 16):
      rows_v[NBUF - 1, i, pl.ds(k * 16, 16)] = zeros16
    return carry

  lax.fori_loop(0, DB, _zero_row, 0)
  zsrc = rows_v.at[NBUF - 1, pl.ds(0, DB)]
  for k in range(KDR):
    blk = sid + k * NS

    @pl.when(blk < NDB)
    def _():
      pltpu.sync_copy(zsrc, acc_sh.at[pl.ds(blk * DB, DB)])

  pltpu.async_copy(table_hbm.at[gidx_v.at[0, NBUF - 1]],
                   rows_v.at[NBUF - 1], gsem.at[NBUF - 1])
  plsc.subcore_barrier()

  def _chunk(j, carry):
    m = lax.rem(j, NBUF)
    grp = lax.div(j, G)
    k = lax.rem(j, G)
    slot = lax.rem(grp, 2)

    @pl.when(jnp.logical_and(k == 0, grp + 1 < NGRP))
    def _():
      pltpu.async_copy(gidx_hbm.at[wid, grp + 1], gidx_v.at[1 - slot],
                       isem)
      pltpu.async_copy(didx_hbm.at[wid, grp + 1], didx_v.at[1 - slot],
                       isem)

    pltpu.make_async_copy(
        table_hbm.at[gidx_v.at[slot, k]], rows_v.at[m], gsem.at[m]).wait()
    pltpu.sync_copy(rows_v.at[m], acc_sh.at[didx_v.at[slot, k]], add=True)

    jn = j + NBUF

    @pl.when(jn < NCH)
    def _():
      kn = lax.rem(jn, G)
      grpn = lax.div(jn, G)
      slotn = lax.rem(grpn, 2)

      @pl.when(kn == 0)
      def _():
        pltpu.make_async_copy(
            gidx_hbm.at[wid, grpn], gidx_v.at[slotn], isem).wait()
        pltpu.make_async_copy(
            didx_hbm.at[wid, grpn], didx_v.at[slotn], isem).wait()

      pltpu.async_copy(table_hbm.at[gidx_v.at[slotn, kn]], rows_v.at[m],
                       gsem.at[m])

    return carry

  lax.fori_loop(0, NCH, _chunk, 0)
  plsc.subcore_barrier()

  for k in range(KDR):
    blk = sid + k * NS

    @pl.when(blk < NDB)
    def _():
      sl = pl.ds(blk * DB, DB)
      pltpu.sync_copy(acc_sh.at[sl], out_hbm.at[cid, sl])


def _sc_scatter(table, gidx, didx):
  mesh = plsc.VectorSubcoreMesh(
      core_axis_name="c", subcore_axis_name="s", num_cores=NC,
      num_subcores=NS)
  f = pl.kernel(
      _sc_scatter_body,
      out_type=jax.ShapeDtypeStruct((NC, N, H), jnp.float32),
      mesh=mesh,
      scratch_types=[
          pltpu.VMEM((2, G, C), jnp.int32),
          pltpu.VMEM((2, G, C), jnp.int32),
          pltpu.VMEM((NBUF, C, H), jnp.float32),
          pltpu.VMEM_SHARED((N, H), jnp.float32),
          pltpu.SemaphoreType.DMA,
          pltpu.SemaphoreType.DMA((NBUF,)),
      ],
  )
  return f(table, gidx, didx)


def kernel(x, edge_index, etype, basis1, coef1, loop1, bias1,
           basis2, coef2, loop2, bias2):
  src = edge_index[0]
  dst = edge_index[1]
  gidx = (etype * N + src).reshape(NW, NGRP, G, C)
  didx = dst.reshape(NW, NGRP, G, C)

  t1 = _project1(x, coef1, basis1, loop1)
  p1 = _sc_scatter(t1.reshape(RP * N, H), gidx, didx)
  t2 = _finish1_project2(p1, t1, bias1.reshape(1, H), coef2, basis2, loop2)
  p2 = _sc_scatter(t2.reshape(RP * N, H), gidx, didx)
  return _finish2(p2, t2, bias2.reshape(1, H))


# speedup vs baseline: 1.0362x; 1.0362x over previous
import jax
import jax.numpy as jnp
from jax import lax
from jax.experimental import pallas as pl
from jax.experimental.pallas import tpu as pltpu
from jax.experimental.pallas import tpu_sc as plsc

N = 10000
E = 320000
D = 128
R = 8
B = 4
H = 128
RP = R + 1

NC = 2
NS = 16
NW = NC * NS
EPW = E // NW
C = 80
NCH = EPW // C
G = 5
NGRP = NCH // G
NBUF = 4
DB = 80
NDB = N // DB
KDR = -(-NDB // NS)

BN = 2000


def _mk_w(coef_ref, basis_ref, r):
  w = coef_ref[r, 0] * basis_ref[0]
  for b in range(1, B):
    w = w + coef_ref[r, b] * basis_ref[b]
  return w


def _project1_body(coef_ref, basis_ref, loop_ref, x_ref, out_ref):
  xb = x_ref[...]
  for r in range(R):
    w = _mk_w(coef_ref, basis_ref, r)
    out_ref[r] = jnp.dot(xb, w, preferred_element_type=jnp.float32)
  out_ref[R] = jnp.dot(xb, loop_ref[...], preferred_element_type=jnp.float32)


def _project1(x, coef1, basis1, loop1):
  return pl.pallas_call(
      _project1_body,
      grid=(N // BN,),
      in_specs=[
          pl.BlockSpec(memory_space=pltpu.SMEM),
          pl.BlockSpec((B, D, H), lambda i: (0, 0, 0)),
          pl.BlockSpec((D, H), lambda i: (0, 0)),
          pl.BlockSpec((BN, D), lambda i: (i, 0)),
      ],
      out_specs=pl.BlockSpec((RP, BN, H), lambda i: (0, i, 0)),
      out_shape=jax.ShapeDtypeStruct((RP, N, H), jnp.float32),
  )(coef1, basis1, loop1, x)


def _finish1_project2_body(coef_ref, basis_ref, loop_ref, p_ref, t1s_ref,
                           bias_ref, out_ref):
  h = p_ref[0] + p_ref[1] + t1s_ref[0] + bias_ref[...]
  h = jnp.where(h >= 0, h, h * jnp.float32(0.01))
  for r in range(R):
    w = _mk_w(coef_ref, basis_ref, r)
    out_ref[r] = jnp.dot(h, w, preferred_element_type=jnp.float32)
  out_ref[R] = jnp.dot(h, loop_ref[...], preferred_element_type=jnp.float32)


def _finish1_project2(p1, t1, bias1, coef2, basis2, loop2):
  return pl.pallas_call(
      _finish1_project2_body,
      grid=(N // BN,),
      in_specs=[
          pl.BlockSpec(memory_space=pltpu.SMEM),
          pl.BlockSpec((B, H, H), lambda i: (0, 0, 0)),
          pl.BlockSpec((H, H), lambda i: (0, 0)),
          pl.BlockSpec((NC, BN, H), lambda i: (0, i, 0)),
          pl.BlockSpec((1, BN, H), lambda i: (R, i, 0)),
          pl.BlockSpec((1, H), lambda i: (0, 0)),
      ],
      out_specs=pl.BlockSpec((RP, BN, H), lambda i: (0, i, 0)),
      out_shape=jax.ShapeDtypeStruct((RP, N, H), jnp.float32),
  )(coef2, basis2, loop2, p1, t1, bias1)


def _finish2_body(p_ref, t2s_ref, bias_ref, out_ref):
  out_ref[...] = p_ref[0] + p_ref[1] + t2s_ref[0] + bias_ref[...]


def _finish2(p2, t2, bias2):
  return pl.pallas_call(
      _finish2_body,
      grid=(N // BN,),
      in_specs=[
          pl.BlockSpec((NC, BN, H), lambda i: (0, i, 0)),
          pl.BlockSpec((1, BN, H), lambda i: (R, i, 0)),
          pl.BlockSpec((1, H), lambda i: (0, 0)),
      ],
      out_specs=pl.BlockSpec((BN, H), lambda i: (i, 0)),
      out_shape=jax.ShapeDtypeStruct((N, H), jnp.float32),
  )(p2, t2, bias2)


def _sc_scatter_body(table_hbm, gidx_hbm, didx_hbm, out_hbm,
                     gidx_v, didx_v, rows_v, acc_sh, isem, gsem):
  cid = lax.axis_index("c")
  sid = lax.axis_index("s")
  wid = sid * NC + cid

  pltpu.async_copy(gidx_hbm.at[wid, 0], gidx_v.at[0], isem)
  pltpu.async_copy(didx_hbm.at[wid, 0], didx_v.at[0], isem)
  pltpu.make_async_copy(gidx_hbm.at[wid, 0], gidx_v.at[0], isem).wait()
  pltpu.make_async_copy(didx_hbm.at[wid, 0], didx_v.at[0], isem).wait()
  for b in range(NBUF - 1):
    pltpu.async_copy(table_hbm.at[gidx_v.at[0, b]], rows_v.at[b],
                     gsem.at[b])

  zeros16 = jnp.zeros((16,), jnp.float32)

  def _zero_row(i, carry):
    for k in range(H // 16):
      rows_v[NBUF - 1, i, pl.ds(k * 16, 16)] = zeros16
    return carry

  lax.fori_loop(0, DB, _zero_row, 0)
  zsrc = rows_v.at[NBUF - 1, pl.ds(0, DB)]
  for k in range(KDR):
    blk = sid + k * NS

    @pl.when(blk < NDB)
    def _():
      pltpu.sync_copy(zsrc, acc_sh.at[pl.ds(blk * DB, DB)])

  pltpu.async_copy(table_hbm.at[gidx_v.at[0, NBUF - 1]],
                   rows_v.at[NBUF - 1], gsem.at[NBUF - 1])
  plsc.subcore_barrier()

  def _chunk(j, carry):
    m = lax.rem(j, NBUF)
    grp = lax.div(j, G)
    k = lax.rem(j, G)
    slot = lax.rem(grp, 2)

    @pl.when(jnp.logical_and(k == 0, grp + 1 < NGRP))
    def _():
      pltpu.async_copy(gidx_hbm.at[wid, grp + 1], gidx_v.at[1 - slot],
                       isem)
      pltpu.async_copy(didx_hbm.at[wid, grp + 1], didx_v.at[1 - slot],
                       isem)

    pltpu.make_async_copy(
        table_hbm.at[gidx_v.at[slot, k]], rows_v.at[m], gsem.at[m]).wait()
    pltpu.sync_copy(rows_v.at[m], acc_sh.at[didx_v.at[slot, k]], add=True)

    jn = j + NBUF

    @pl.when(jn < NCH)
    def _():
      kn = lax.rem(jn, G)
      grpn = lax.div(jn, G)
      slotn = lax.rem(grpn, 2)

      @pl.when(kn == 0)
      def _():
        pltpu.make_async_copy(
            gidx_hbm.at[wid, grpn], gidx_v.at[slotn], isem).wait()
        pltpu.make_async_copy(
            didx_hbm.at[wid, grpn], didx_v.at[slotn], isem).wait()

      pltpu.async_copy(table_hbm.at[gidx_v.at[slotn, kn]], rows_v.at[m],
                       gsem.at[m])

    return carry

  lax.fori_loop(0, NCH, _chunk, 0)
  plsc.subcore_barrier()

  for k in range(KDR):
    blk = sid + k * NS

    @pl.when(blk < NDB)
    def _():
      sl = pl.ds(blk * DB, DB)
      pltpu.async_copy(acc_sh.at[sl], out_hbm.at[cid, sl], isem)

  for k in range(KDR):
    blk = sid + k * NS

    @pl.when(blk < NDB)
    def _():
      sl = pl.ds(blk * DB, DB)
      pltpu.make_async_copy(acc_sh.at[sl], out_hbm.at[cid, sl],
                            isem).wait()


def _sc_scatter(table, gidx, didx):
  mesh = plsc.VectorSubcoreMesh(
      core_axis_name="c", subcore_axis_name="s", num_cores=NC,
      num_subcores=NS)
  f = pl.kernel(
      _sc_scatter_body,
      out_type=jax.ShapeDtypeStruct((NC, N, H), jnp.float32),
      mesh=mesh,
      scratch_types=[
          pltpu.VMEM((2, G, C), jnp.int32),
          pltpu.VMEM((2, G, C), jnp.int32),
          pltpu.VMEM((NBUF, C, H), jnp.float32),
          pltpu.VMEM_SHARED((N, H), jnp.float32),
          pltpu.SemaphoreType.DMA,
          pltpu.SemaphoreType.DMA((NBUF,)),
      ],
  )
  return f(table, gidx, didx)


def kernel(x, edge_index, etype, basis1, coef1, loop1, bias1,
           basis2, coef2, loop2, bias2):
  src = edge_index[0]
  dst = edge_index[1]
  gidx = (etype * N + src).reshape(NW, NGRP, G, C)
  didx = dst.reshape(NW, NGRP, G, C)

  t1 = _project1(x, coef1, basis1, loop1)
  p1 = _sc_scatter(t1.reshape(RP * N, H), gidx, didx)
  t2 = _finish1_project2(p1, t1, bias1.reshape(1, H), coef2, basis2, loop2)
  p2 = _sc_scatter(t2.reshape(RP * N, H), gidx, didx)
  return _finish2(p2, t2, bias2.reshape(1, H))
